# baseline (speedup 1.0000x reference)
import functools

import jax
import jax.numpy as jnp
from jax import lax
from jax.experimental import pallas as pl
from jax.experimental.pallas import tpu as pltpu
from jax.experimental.pallas import tpu_sc as plsc

N = 10000
D = 128
E = 320000
NUM_LAYERS = 3

NC, NS = 2, 16
N_TILES = NC * NS
CH = 128
CHUNKS_PER_TILE = 80
GCH = 16
C0_CHUNKS = 32
C1_CHUNKS = 128
E_PAD = N_TILES * CHUNKS_PER_TILE * CH
N_CHUNKS = E_PAD // CH
N_PAD = 10112
DUMP_ROW = N
ZROWS = N_PAD // NS

_HI = jax.lax.Precision.HIGHEST


def _sc_agg_body(x_hbm, src_hbm, dst_hbm, z_hbm, out_hbm,
                 srcv, dstv, buf0, buf1, sem0, sem1, agg):
    cid = lax.axis_index("c")
    sid = lax.axis_index("s")
    tile = cid * NS + sid

    pltpu.sync_copy(z_hbm, agg.at[pl.ds(sid * ZROWS, ZROWS)])
    plsc.subcore_barrier()

    def run_edges(chunk_base, nchunks):
        for g in range(nchunks // GCH):
            base = chunk_base + g * GCH
            pltpu.sync_copy(src_hbm.at[pl.ds(base, GCH)], srcv)
            pltpu.sync_copy(dst_hbm.at[pl.ds(base, GCH)], dstv)

            pltpu.async_copy(x_hbm.at[srcv.at[0]], buf0, sem0)
            pltpu.async_copy(x_hbm.at[srcv.at[1]], buf1, sem1)

            @pl.loop(0, GCH // 2 - 1)
            def _(j2):
                j = j2 * 2
                pltpu.make_async_copy(x_hbm.at[srcv.at[j]], buf0, sem0).wait()
                pltpu.sync_copy(buf0, agg.at[dstv.at[j]], add=True)
                pltpu.async_copy(x_hbm.at[srcv.at[j + 2]], buf0, sem0)
                pltpu.make_async_copy(x_hbm.at[srcv.at[j + 1]], buf1, sem1).wait()
                pltpu.sync_copy(buf1, agg.at[dstv.at[j + 1]], add=True)
                pltpu.async_copy(x_hbm.at[srcv.at[j + 3]], buf1, sem1)

            jl = GCH - 2
            pltpu.make_async_copy(x_hbm.at[srcv.at[jl]], buf0, sem0).wait()
            pltpu.sync_copy(buf0, agg.at[dstv.at[jl]], add=True)
            pltpu.make_async_copy(x_hbm.at[srcv.at[jl + 1]], buf1, sem1).wait()
            pltpu.sync_copy(buf1, agg.at[dstv.at[jl + 1]], add=True)

    @pl.when(cid == 0)
    def _():
        run_edges(sid * C0_CHUNKS, C0_CHUNKS)

    @pl.when(cid == 1)
    def _():
        run_edges(NS * C0_CHUNKS + sid * C1_CHUNKS, C1_CHUNKS)

    plsc.subcore_barrier()
    pltpu.sync_copy(agg.at[pl.ds(sid * ZROWS, ZROWS)],
                    out_hbm.at[cid, pl.ds(sid * ZROWS, ZROWS)])


@functools.lru_cache(maxsize=None)
def _make_sc_aggregate():
    return pl.kernel(
        _sc_agg_body,
        out_type=jax.ShapeDtypeStruct((NC, N_PAD, D), jnp.float32),
        mesh=plsc.VectorSubcoreMesh(core_axis_name="c", subcore_axis_name="s",
                                    num_cores=NC, num_subcores=NS),
        scratch_types=[
            pltpu.VMEM((GCH, CH), jnp.int32),
            pltpu.VMEM((GCH, CH), jnp.int32),
            pltpu.VMEM((CH, D), jnp.float32),
            pltpu.VMEM((CH, D), jnp.float32),
            pltpu.SemaphoreType.DMA,
            pltpu.SemaphoreType.DMA,
            pltpu.VMEM_SHARED((N_PAD, D), jnp.float32),
        ],
    )


def _sc_aggregate(*args):
    return _make_sc_aggregate()(*args)


def _roll(v, shift):
    return pltpu.roll(v, shift % 128, axis=1)


def _bitonic128(v):
    lane = lax.broadcasted_iota(jnp.int32, v.shape, 1)
    k = 2
    while k <= 128:
        j = k // 2
        while j >= 1:
            upper = (lane & j) != 0
            pa = jnp.where(upper, _roll(v, j), _roll(v, -j))
            keep_min = jnp.logical_not(upper) == ((lane & k) == 0)
            v = jnp.where(keep_min, jnp.minimum(v, pa), jnp.maximum(v, pa))
            j //= 2
        k *= 2
    return v


def _tc_layer_body(x_ref, n_ref, wm_ref, ws_ref, g_ref, b_ref,
                   wh_ref, bh_ref, o_ref, *, residual, head):
    ns = n_ref[0] + n_ref[1]
    t = lax.dot(ns, wm_ref[...], precision=_HI,
                preferred_element_type=jnp.float32)
    h = _bitonic128(t)
    h = h + lax.dot(x_ref[...], ws_ref[...], precision=_HI,
                    preferred_element_type=jnp.float32)
    mu = jnp.mean(h, axis=-1, keepdims=True)
    c = h - mu
    var = jnp.mean(c * c, axis=-1, keepdims=True)
    h = c * lax.rsqrt(var + 1e-5) * g_ref[...] + b_ref[...]
    if residual:
        h = h + x_ref[...]
    if head:
        h = lax.dot(h, wh_ref[...], precision=_HI,
                    preferred_element_type=jnp.float32) + bh_ref[...]
    o_ref[...] = h


def _tc_layer(x, n, wm, ws, g, b, wh, bh, *, residual, head, block_rows=1000):
    grid = N // block_rows
    full = lambda i: (0, 0)
    return pl.pallas_call(
        functools.partial(_tc_layer_body, residual=residual, head=head),
        grid=(grid,),
        in_specs=[
            pl.BlockSpec((block_rows, D), lambda i: (i, 0)),
            pl.BlockSpec((NC, block_rows, D), lambda i: (0, i, 0)),
            pl.BlockSpec((D, D), full),
            pl.BlockSpec((D, D), full),
            pl.BlockSpec((1, D), full),
            pl.BlockSpec((1, D), full),
            pl.BlockSpec((D, D), full),
            pl.BlockSpec((1, D), full),
        ],
        out_specs=pl.BlockSpec((block_rows, D), lambda i: (i, 0)),
        out_shape=jax.ShapeDtypeStruct((N, D), jnp.float32),
    )(x, n, wm, ws, g, b, wh, bh)


def kernel(x, edge_index, Wm, Ws, gamma, beta, W_head, b_head):
    src = edge_index[0].astype(jnp.int32)
    dst = edge_index[1].astype(jnp.int32)
    pad = E_PAD - E
    src_p = jnp.concatenate([src, jnp.zeros((pad,), jnp.int32)]).reshape(N_CHUNKS, CH)
    dst_p = jnp.concatenate([dst, jnp.full((pad,), DUMP_ROW, jnp.int32)]).reshape(N_CHUNKS, CH)
    z = jnp.zeros((ZROWS, D), jnp.float32)
    g2 = gamma.reshape(NUM_LAYERS, 1, D)
    b2 = beta.reshape(NUM_LAYERS, 1, D)
    bh2 = b_head.reshape(1, D)

    h = x
    for i in range(NUM_LAYERS):
        npart = _sc_aggregate(h, src_p, dst_p, z)
        h = _tc_layer(h, npart, Wm[i], Ws[i], g2[i], b2[i], W_head, bh2,
                      residual=(i > 0), head=(i == NUM_LAYERS - 1))
    return h


# speedup vs baseline: 1.1584x; 1.1584x over previous
import functools

import jax
import jax.numpy as jnp
from jax import lax
from jax.experimental import pallas as pl
from jax.experimental.pallas import tpu as pltpu
from jax.experimental.pallas import tpu_sc as plsc

N = 10000
D = 128
E = 320000
NUM_LAYERS = 3

NC, NS = 2, 16
N_TILES = NC * NS
CH = 128
CHUNKS_PER_TILE = 80
GCH = 16
C0_CHUNKS = 128
C1_CHUNKS = 32
E_PAD = N_TILES * CHUNKS_PER_TILE * CH
N_CHUNKS = E_PAD // CH
N_PAD = 10112
DUMP_ROW = N
ZROWS = N_PAD // NS

_HI = jax.lax.Precision.HIGHEST


def _sc_agg_body(x_hbm, src_hbm, dst_hbm, z_hbm, out_hbm,
                 srcv, dstv, buf0, buf1, sem0, sem1, agg):
    cid = lax.axis_index("c")
    sid = lax.axis_index("s")
    tile = cid * NS + sid

    pltpu.sync_copy(z_hbm, agg.at[pl.ds(sid * ZROWS, ZROWS)])
    plsc.subcore_barrier()

    def run_edges(chunk_base, nchunks):
        for g in range(nchunks // GCH):
            base = chunk_base + g * GCH
            pltpu.sync_copy(src_hbm.at[pl.ds(base, GCH)], srcv)
            pltpu.sync_copy(dst_hbm.at[pl.ds(base, GCH)], dstv)

            pltpu.async_copy(x_hbm.at[srcv.at[0]], buf0, sem0)
            pltpu.async_copy(x_hbm.at[srcv.at[1]], buf1, sem1)

            @pl.loop(0, GCH // 2 - 1)
            def _(j2):
                j = j2 * 2
                pltpu.make_async_copy(x_hbm.at[srcv.at[j]], buf0, sem0).wait()
                pltpu.sync_copy(buf0, agg.at[dstv.at[j]], add=True)
                pltpu.async_copy(x_hbm.at[srcv.at[j + 2]], buf0, sem0)
                pltpu.make_async_copy(x_hbm.at[srcv.at[j + 1]], buf1, sem1).wait()
                pltpu.sync_copy(buf1, agg.at[dstv.at[j + 1]], add=True)
                pltpu.async_copy(x_hbm.at[srcv.at[j + 3]], buf1, sem1)

            jl = GCH - 2
            pltpu.make_async_copy(x_hbm.at[srcv.at[jl]], buf0, sem0).wait()
            pltpu.sync_copy(buf0, agg.at[dstv.at[jl]], add=True)
            pltpu.make_async_copy(x_hbm.at[srcv.at[jl + 1]], buf1, sem1).wait()
            pltpu.sync_copy(buf1, agg.at[dstv.at[jl + 1]], add=True)

    @pl.when(cid == 0)
    def _():
        run_edges(sid * C0_CHUNKS, C0_CHUNKS)

    @pl.when(cid == 1)
    def _():
        run_edges(NS * C0_CHUNKS + sid * C1_CHUNKS, C1_CHUNKS)

    plsc.subcore_barrier()
    pltpu.sync_copy(agg.at[pl.ds(sid * ZROWS, ZROWS)],
                    out_hbm.at[cid, pl.ds(sid * ZROWS, ZROWS)])


@functools.lru_cache(maxsize=None)
def _make_sc_aggregate():
    return pl.kernel(
        _sc_agg_body,
        out_type=jax.ShapeDtypeStruct((NC, N_PAD, D), jnp.float32),
        mesh=plsc.VectorSubcoreMesh(core_axis_name="c", subcore_axis_name="s",
                                    num_cores=NC, num_subcores=NS),
        scratch_types=[
            pltpu.VMEM((GCH, CH), jnp.int32),
            pltpu.VMEM((GCH, CH), jnp.int32),
            pltpu.VMEM((CH, D), jnp.float32),
            pltpu.VMEM((CH, D), jnp.float32),
            pltpu.SemaphoreType.DMA,
            pltpu.SemaphoreType.DMA,
            pltpu.VMEM_SHARED((N_PAD, D), jnp.float32),
        ],
    )


def _sc_aggregate(*args):
    return _make_sc_aggregate()(*args)


def _roll(v, shift):
    return pltpu.roll(v, shift % 128, axis=1)


def _bitonic128(v):
    lane = lax.broadcasted_iota(jnp.int32, v.shape, 1)
    k = 2
    while k <= 128:
        j = k // 2
        while j >= 1:
            upper = (lane & j) != 0
            pa = jnp.where(upper, _roll(v, j), _roll(v, -j))
            keep_min = jnp.logical_not(upper) == ((lane & k) == 0)
            v = jnp.where(keep_min, jnp.minimum(v, pa), jnp.maximum(v, pa))
            j //= 2
        k *= 2
    return v


def _tc_layer_body(x_ref, n_ref, wm_ref, ws_ref, g_ref, b_ref,
                   wh_ref, bh_ref, o_ref, *, residual, head):
    ns = n_ref[0] + n_ref[1]
    t = lax.dot(ns, wm_ref[...], precision=_HI,
                preferred_element_type=jnp.float32)
    h = _bitonic128(t)
    h = h + lax.dot(x_ref[...], ws_ref[...], precision=_HI,
                    preferred_element_type=jnp.float32)
    mu = jnp.mean(h, axis=-1, keepdims=True)
    c = h - mu
    var = jnp.mean(c * c, axis=-1, keepdims=True)
    h = c * lax.rsqrt(var + 1e-5) * g_ref[...] + b_ref[...]
    if residual:
        h = h + x_ref[...]
    if head:
        h = lax.dot(h, wh_ref[...], precision=_HI,
                    preferred_element_type=jnp.float32) + bh_ref[...]
    o_ref[...] = h


def _tc_layer(x, n, wm, ws, g, b, wh, bh, *, residual, head, block_rows=1000):
    grid = N // block_rows
    full = lambda i: (0, 0)
    return pl.pallas_call(
        functools.partial(_tc_layer_body, residual=residual, head=head),
        grid=(grid,),
        in_specs=[
            pl.BlockSpec((block_rows, D), lambda i: (i, 0)),
            pl.BlockSpec((NC, block_rows, D), lambda i: (0, i, 0)),
            pl.BlockSpec((D, D), full),
            pl.BlockSpec((D, D), full),
            pl.BlockSpec((1, D), full),
            pl.BlockSpec((1, D), full),
            pl.BlockSpec((D, D), full),
            pl.BlockSpec((1, D), full),
        ],
        out_specs=pl.BlockSpec((block_rows, D), lambda i: (i, 0)),
        out_shape=jax.ShapeDtypeStruct((N, D), jnp.float32),
    )(x, n, wm, ws, g, b, wh, bh)


def kernel(x, edge_index, Wm, Ws, gamma, beta, W_head, b_head):
    src = edge_index[0].astype(jnp.int32)
    dst = edge_index[1].astype(jnp.int32)
    pad = E_PAD - E
    src_p = jnp.concatenate([src, jnp.zeros((pad,), jnp.int32)]).reshape(N_CHUNKS, CH)
    dst_p = jnp.concatenate([dst, jnp.full((pad,), DUMP_ROW, jnp.int32)]).reshape(N_CHUNKS, CH)
    z = jnp.zeros((ZROWS, D), jnp.float32)
    g2 = gamma.reshape(NUM_LAYERS, 1, D)
    b2 = beta.reshape(NUM_LAYERS, 1, D)
    bh2 = b_head.reshape(1, D)

    h = x
    for i in range(NUM_LAYERS):
        npart = _sc_aggregate(h, src_p, dst_p, z)
        h = _tc_layer(h, npart, Wm[i], Ws[i], g2[i], b2[i], W_head, bh2,
                      residual=(i > 0), head=(i == NUM_LAYERS - 1))
    return h
